# Initial kernel scaffold; baseline (speedup 1.0000x reference)
import jax
import jax.numpy as jnp
from jax.experimental import pallas as pl


def kernel(x, edge_index, W, b, W2, b2):
    raise NotImplementedError("write your pallas kernel here")


# speedup vs baseline: 26.0321x; 26.0321x over previous
import functools

import jax
import jax.numpy as jnp
from jax import lax
from jax.experimental import pallas as pl
from jax.experimental.pallas import tpu as pltpu
from jax.experimental.pallas import tpu_sc as plsc

N = 10000
D = 128
E = 320000
NC, NS = 2, 16
NW = NC * NS
EW = E // NW
CH = 80
NCHUNK = EW // CH
RPT = 624
TAIL = N - NS * RPT
RB = 1000

_mesh = plsc.VectorSubcoreMesh(core_axis_name="c", subcore_axis_name="s")


@functools.partial(
    pl.kernel,
    out_type=jax.ShapeDtypeStruct((NW * N,), jnp.float32),
    mesh=_mesh,
    compiler_params=pltpu.CompilerParams(needs_layout_passes=False),
    scratch_types=[
        pltpu.VMEM((EW,), jnp.int32),
        pltpu.VMEM((N,), jnp.float32),
    ],
)
def _deg_kernel(dst_hbm, out_hbm, idx_v, deg_v):
    c = lax.axis_index("c")
    s = lax.axis_index("s")
    w = s * NC + c

    zeros16 = jnp.zeros((16,), jnp.float32)

    def zbody(i, carry):
        deg_v[pl.ds(i * 16, 16)] = zeros16
        return carry

    lax.fori_loop(0, N // 16, zbody, 0)

    off = pl.multiple_of(w * EW, 8)
    pltpu.sync_copy(dst_hbm.at[pl.ds(off, EW)], idx_v)

    ones16 = jnp.ones((16,), jnp.float32)

    def body(i, carry):
        idx = idx_v[pl.ds(i * 16, 16)]
        plsc.addupdate_scatter(deg_v, [idx], ones16)
        return carry

    lax.fori_loop(0, EW // 16, body, 0)
    oof = pl.multiple_of(w * N, 8)
    pltpu.sync_copy(deg_v, out_hbm.at[pl.ds(oof, N)])


@functools.partial(
    pl.kernel,
    out_type=jax.ShapeDtypeStruct((NC, N, D), jnp.float32),
    mesh=_mesh,
    compiler_params=pltpu.CompilerParams(needs_layout_passes=False),
    scratch_types=[
        pltpu.VMEM((CH,), jnp.int32),
        pltpu.VMEM((CH,), jnp.int32),
        pltpu.VMEM((CH,), jnp.int32),
        pltpu.VMEM((CH,), jnp.int32),
        pltpu.VMEM((CH, D), jnp.float32),
        pltpu.VMEM((CH, D), jnp.float32),
        pltpu.VMEM_SHARED((N, D), jnp.float32),
        pltpu.SemaphoreType.DMA,
        pltpu.SemaphoreType.DMA,
    ],
)
def _edge_kernel(hp_hbm, src_hbm, dst_hbm, out_hbm,
                 sidx0, sidx1, didx0, didx1, rows0, rows1, acc, sem0, sem1):
    c = lax.axis_index("c")
    s = lax.axis_index("s")
    w = s * NC + c
    ebase = w * EW
    rbase = s * RPT

    zeros16 = jnp.zeros((16,), jnp.float32)

    def zr(i, carry):
        rows0[i // 8, pl.ds((i % 8) * 16, 16)] = zeros16
        return carry

    lax.fori_loop(0, CH * 8, zr, 0)

    def zacc(k, carry):
        pltpu.sync_copy(rows0, acc.at[pl.ds(rbase + k * CH, CH)])
        return carry

    lax.fori_loop(0, RPT // CH, zacc, 0)
    rem = RPT - (RPT // CH) * CH
    pltpu.sync_copy(rows0.at[pl.ds(0, rem)],
                    acc.at[pl.ds(rbase + RPT - rem, rem)])

    @pl.when(s == NS - 1)
    def _zero_tail():
        pltpu.sync_copy(rows0.at[pl.ds(0, TAIL)],
                        acc.at[pl.ds(NS * RPT, TAIL)])

    plsc.subcore_barrier()

    def load_sidx(buf, ci):
        off = pl.multiple_of(ebase + ci * CH, 8)
        pltpu.sync_copy(src_hbm.at[pl.ds(off, CH)], buf)

    def load_didx(buf, ci):
        off = pl.multiple_of(ebase + ci * CH, 8)
        pltpu.sync_copy(dst_hbm.at[pl.ds(off, CH)], buf)

    load_sidx(sidx0, 0)
    pltpu.async_copy(hp_hbm.at[sidx0], rows0, sem0)

    def body(t, carry):
        c0 = 2 * t
        load_sidx(sidx1, c0 + 1)
        pltpu.async_copy(hp_hbm.at[sidx1], rows1, sem1)
        pltpu.make_async_copy(hp_hbm.at[sidx0], rows0, sem0).wait()
        load_didx(didx0, c0)
        pltpu.sync_copy(rows0, acc.at[didx0], add=True)

        load_sidx(sidx0, c0 + 2)
        pltpu.async_copy(hp_hbm.at[sidx0], rows0, sem0)
        pltpu.make_async_copy(hp_hbm.at[sidx1], rows1, sem1).wait()
        load_didx(didx1, c0 + 1)
        pltpu.sync_copy(rows1, acc.at[didx1], add=True)
        return carry

    lax.fori_loop(0, (NCHUNK - 1) // 2, body, 0)

    pltpu.make_async_copy(hp_hbm.at[sidx0], rows0, sem0).wait()
    load_didx(didx0, NCHUNK - 1)
    pltpu.sync_copy(rows0, acc.at[didx0], add=True)

    plsc.subcore_barrier()
    pltpu.sync_copy(acc.at[pl.ds(rbase, RPT)],
                    out_hbm.at[c, pl.ds(rbase, RPT)])

    @pl.when(s == NS - 1)
    def _copy_tail():
        pltpu.sync_copy(acc.at[pl.ds(NS * RPT, TAIL)],
                        out_hbm.at[c, pl.ds(NS * RPT, TAIL)])


def _dis_body(degp_ref, dis_ref):
    deg = jnp.sum(degp_ref[...], axis=0)
    dis = jnp.where(deg > 0, lax.rsqrt(jnp.maximum(deg, 1e-12)), 0.0)
    dis_ref[...] = dis[:, None]


def _tc_dis(degp):
    return pl.pallas_call(
        _dis_body,
        out_shape=jax.ShapeDtypeStruct((N, 1), jnp.float32),
    )(degp)


def _prep_body(x_ref, w_ref, dis_ref, hp_ref):
    h = jnp.dot(x_ref[...], w_ref[...], preferred_element_type=jnp.float32)
    hp_ref[...] = h * dis_ref[...]


def _tc_prep(x, W, dis):
    return pl.pallas_call(
        _prep_body,
        grid=(N // RB,),
        in_specs=[
            pl.BlockSpec((RB, D), lambda i: (i, 0)),
            pl.BlockSpec((D, D), lambda i: (0, 0)),
            pl.BlockSpec((RB, 1), lambda i: (i, 0)),
        ],
        out_specs=pl.BlockSpec((RB, D), lambda i: (i, 0)),
        out_shape=jax.ShapeDtypeStruct((N, D), jnp.float32),
    )(x, W, dis)


def _final_body(p0_ref, p1_ref, dis_ref, b_ref, w2_ref, b2_ref, out_ref):
    agg = (p0_ref[...] + p1_ref[...]) * dis_ref[...]
    hidden = jnp.maximum(agg + b_ref[...], 0.0)
    out_ref[...] = (jnp.sum(hidden * w2_ref[...], axis=1, keepdims=True)
                    + b2_ref[0, 0])


def _tc_final(p0, p1, dis, b, w2, b2):
    return pl.pallas_call(
        _final_body,
        grid=(N // RB,),
        in_specs=[
            pl.BlockSpec((RB, D), lambda i: (i, 0)),
            pl.BlockSpec((RB, D), lambda i: (i, 0)),
            pl.BlockSpec((RB, 1), lambda i: (i, 0)),
            pl.BlockSpec((1, D), lambda i: (0, 0)),
            pl.BlockSpec((1, D), lambda i: (0, 0)),
            pl.BlockSpec((1, 1), lambda i: (0, 0)),
        ],
        out_specs=pl.BlockSpec((RB, 1), lambda i: (i, 0)),
        out_shape=jax.ShapeDtypeStruct((N, 1), jnp.float32),
    )(p0, p1, dis, b, w2, b2)


def kernel(x, edge_index, W, b, W2, b2):
    ei = edge_index.astype(jnp.int32)
    src = ei[0]
    dst = ei[1]
    degp = _deg_kernel(dst).reshape(NW, N)
    dis = _tc_dis(degp)
    hp = _tc_prep(x, W, dis)
    paggs = _edge_kernel(hp, src, dst)
    return _tc_final(paggs[0], paggs[1], dis,
                     b.reshape(1, D), W2.reshape(1, D), b2.reshape(1, 1))


# speedup vs baseline: 33.2805x; 1.2784x over previous
import functools

import jax
import jax.numpy as jnp
from jax import lax
from jax.experimental import pallas as pl
from jax.experimental.pallas import tpu as pltpu
from jax.experimental.pallas import tpu_sc as plsc

N = 10000
D = 128
E = 320000
NC, NS = 2, 16
NW = NC * NS
EW = E // NW
CH = 128
NCHUNK = EW // CH
TAILE = EW - NCHUNK * CH
NBUF = 3
RPT = 624
TAIL = N - NS * RPT
RB = 1000

_mesh = plsc.VectorSubcoreMesh(core_axis_name="c", subcore_axis_name="s")


@functools.partial(
    pl.kernel,
    out_type=jax.ShapeDtypeStruct((NW * N,), jnp.float32),
    mesh=_mesh,
    compiler_params=pltpu.CompilerParams(needs_layout_passes=False),
    scratch_types=[
        pltpu.VMEM((EW,), jnp.int32),
        pltpu.VMEM((N,), jnp.float32),
    ],
)
def _deg_kernel(dst_hbm, out_hbm, idx_v, deg_v):
    c = lax.axis_index("c")
    s = lax.axis_index("s")
    w = s * NC + c

    zeros16 = jnp.zeros((16,), jnp.float32)

    def zbody(i, carry):
        deg_v[pl.ds(i * 16, 16)] = zeros16
        return carry

    lax.fori_loop(0, N // 16, zbody, 0)

    off = pl.multiple_of(w * EW, 8)
    pltpu.sync_copy(dst_hbm.at[pl.ds(off, EW)], idx_v)

    ones16 = jnp.ones((16,), jnp.float32)

    def body(i, carry):
        idx = idx_v[pl.ds(i * 16, 16)]
        plsc.addupdate_scatter(deg_v, [idx], ones16)
        return carry

    lax.fori_loop(0, EW // 16, body, 0)
    oof = pl.multiple_of(w * N, 8)
    pltpu.sync_copy(deg_v, out_hbm.at[pl.ds(oof, N)])


@functools.partial(
    pl.kernel,
    out_type=jax.ShapeDtypeStruct((NC, N, D), jnp.float32),
    mesh=_mesh,
    compiler_params=pltpu.CompilerParams(needs_layout_passes=False),
    scratch_types=[
        [pltpu.VMEM((CH,), jnp.int32)] * NBUF,
        [pltpu.VMEM((CH,), jnp.int32)] * NBUF,
        pltpu.VMEM((TAILE,), jnp.int32),
        pltpu.VMEM((TAILE,), jnp.int32),
        [pltpu.VMEM((CH, D), jnp.float32)] * NBUF,
        pltpu.VMEM_SHARED((N, D), jnp.float32),
        [pltpu.SemaphoreType.DMA] * NBUF,
        [pltpu.SemaphoreType.DMA] * NBUF,
    ],
)
def _edge_kernel(hp_hbm, src_hbm, dst_hbm, out_hbm,
                 sidx, didx, sidx_t, didx_t, rows, acc, gsem, ssem):
    c = lax.axis_index("c")
    s = lax.axis_index("s")
    w = s * NC + c
    ebase = pl.multiple_of(w * EW, 8)
    rbase = s * RPT

    zeros16 = jnp.zeros((16,), jnp.float32)

    def zr(i, carry):
        rows[0][i // 8, pl.ds((i % 8) * 16, 16)] = zeros16
        return carry

    lax.fori_loop(0, CH * 8, zr, 0)

    def zacc(k, carry):
        pltpu.sync_copy(rows[0], acc.at[pl.ds(rbase + k * CH, CH)])
        return carry

    lax.fori_loop(0, RPT // CH, zacc, 0)
    rem = RPT - (RPT // CH) * CH
    pltpu.sync_copy(rows[0].at[pl.ds(0, rem)],
                    acc.at[pl.ds(rbase + RPT - rem, rem)])

    @pl.when(s == NS - 1)
    def _zero_tail():
        pltpu.sync_copy(rows[0].at[pl.ds(0, TAIL)],
                        acc.at[pl.ds(NS * RPT, TAIL)])

    plsc.subcore_barrier()

    def load_sidx(b, ci):
        off = pl.multiple_of(ebase + ci * CH, 8)
        pltpu.sync_copy(src_hbm.at[pl.ds(off, CH)], sidx[b])

    def start_gather(b):
        pltpu.async_copy(hp_hbm.at[sidx[b]], rows[b], gsem[b])

    def wait_gather(b):
        pltpu.make_async_copy(hp_hbm.at[sidx[b]], rows[b], gsem[b]).wait()

    def load_didx(b, ci):
        off = pl.multiple_of(ebase + ci * CH, 8)
        pltpu.sync_copy(dst_hbm.at[pl.ds(off, CH)], didx[b])

    def start_scatter(b):
        pltpu.async_copy(rows[b], acc.at[didx[b]], ssem[b], add=True)

    def wait_scatter(b):
        pltpu.make_async_copy(rows[b], acc.at[didx[b]], ssem[b]).wait()

    for b in range(NBUF):
        load_sidx(b, b)
        start_gather(b)

    def body(t, carry):
        c0 = t * NBUF
        for b in range(NBUF):
            wait_gather(b)
            load_didx(b, c0 + b)
            start_scatter(b)
        for b in range(NBUF):
            wait_scatter(b)
            load_sidx(b, c0 + NBUF + b)
            start_gather(b)
        return carry

    lax.fori_loop(0, NCHUNK // NBUF - 1, body, 0)

    clast = NCHUNK - NBUF
    for b in range(NBUF):
        wait_gather(b)
        load_didx(b, clast + b)
        start_scatter(b)
    for b in range(NBUF):
        wait_scatter(b)

    toff = pl.multiple_of(ebase + NCHUNK * CH, 8)
    pltpu.sync_copy(src_hbm.at[pl.ds(toff, TAILE)], sidx_t)
    pltpu.async_copy(hp_hbm.at[sidx_t], rows[0].at[pl.ds(0, TAILE)],
                     gsem[0]).wait()
    pltpu.sync_copy(dst_hbm.at[pl.ds(toff, TAILE)], didx_t)
    pltpu.sync_copy(rows[0].at[pl.ds(0, TAILE)], acc.at[didx_t], add=True)

    plsc.subcore_barrier()
    pltpu.sync_copy(acc.at[pl.ds(rbase, RPT)],
                    out_hbm.at[c, pl.ds(rbase, RPT)])

    @pl.when(s == NS - 1)
    def _copy_tail():
        pltpu.sync_copy(acc.at[pl.ds(NS * RPT, TAIL)],
                        out_hbm.at[c, pl.ds(NS * RPT, TAIL)])


def _prep_body(x_ref, w_ref, degp_ref, hp_ref, dis_ref):
    deg = jnp.sum(degp_ref[...], axis=0)
    dis = jnp.where(deg > 0, lax.rsqrt(jnp.maximum(deg, 1e-12)), 0.0)
    h = jnp.dot(x_ref[...], w_ref[...], preferred_element_type=jnp.float32)
    hp_ref[...] = h * dis[:, None]
    dis_ref[...] = dis[:, None]


def _tc_prep(x, W, degp):
    return pl.pallas_call(
        _prep_body,
        out_shape=[
            jax.ShapeDtypeStruct((N, D), jnp.float32),
            jax.ShapeDtypeStruct((N, 1), jnp.float32),
        ],
    )(x, W, degp)


def _final_body(p0_ref, p1_ref, dis_ref, b_ref, w2_ref, b2_ref, out_ref):
    agg = (p0_ref[...] + p1_ref[...]) * dis_ref[...]
    hidden = jnp.maximum(agg + b_ref[...], 0.0)
    out_ref[...] = (jnp.sum(hidden * w2_ref[...], axis=1, keepdims=True)
                    + b2_ref[0, 0])


def _tc_final(p0, p1, dis, b, w2, b2):
    return pl.pallas_call(
        _final_body,
        grid=(N // RB,),
        in_specs=[
            pl.BlockSpec((RB, D), lambda i: (i, 0)),
            pl.BlockSpec((RB, D), lambda i: (i, 0)),
            pl.BlockSpec((RB, 1), lambda i: (i, 0)),
            pl.BlockSpec((1, D), lambda i: (0, 0)),
            pl.BlockSpec((1, D), lambda i: (0, 0)),
            pl.BlockSpec((1, 1), lambda i: (0, 0)),
        ],
        out_specs=pl.BlockSpec((RB, 1), lambda i: (i, 0)),
        out_shape=jax.ShapeDtypeStruct((N, 1), jnp.float32),
    )(p0, p1, dis, b, w2, b2)


def kernel(x, edge_index, W, b, W2, b2):
    ei = edge_index.astype(jnp.int32)
    src = ei[0]
    dst = ei[1]
    degp = _deg_kernel(dst).reshape(NW, N)
    hp, dis = _tc_prep(x, W, degp)
    paggs = _edge_kernel(hp, src, dst)
    return _tc_final(paggs[0], paggs[1], dis,
                     b.reshape(1, D), W2.reshape(1, D), b2.reshape(1, 1))


# speedup vs baseline: 33.9005x; 1.0186x over previous
import functools

import jax
import jax.numpy as jnp
from jax import lax
from jax.experimental import pallas as pl
from jax.experimental.pallas import tpu as pltpu
from jax.experimental.pallas import tpu_sc as plsc

N = 10000
D = 128
E = 320000
NC, NS = 2, 16
NW = NC * NS
EW = E // NW
CH = 104
NCHUNK = EW // CH
TAILE = EW - NCHUNK * CH
NBUF = 3
NITER = NCHUNK // NBUF
RPT = 624
TAIL = N - NS * RPT
RB = 1000

_mesh = plsc.VectorSubcoreMesh(core_axis_name="c", subcore_axis_name="s")


@functools.partial(
    pl.kernel,
    out_type=jax.ShapeDtypeStruct((NW * N,), jnp.float32),
    mesh=_mesh,
    compiler_params=pltpu.CompilerParams(needs_layout_passes=False),
    scratch_types=[
        pltpu.VMEM((EW,), jnp.int32),
        pltpu.VMEM((N,), jnp.float32),
    ],
)
def _deg_kernel(dst_hbm, out_hbm, idx_v, deg_v):
    c = lax.axis_index("c")
    s = lax.axis_index("s")
    w = s * NC + c

    zeros16 = jnp.zeros((16,), jnp.float32)

    def zbody(i, carry):
        deg_v[pl.ds(i * 16, 16)] = zeros16
        return carry

    lax.fori_loop(0, N // 16, zbody, 0)

    off = pl.multiple_of(w * EW, 8)
    pltpu.sync_copy(dst_hbm.at[pl.ds(off, EW)], idx_v)

    ones16 = jnp.ones((16,), jnp.float32)

    def body(i, carry):
        idx = idx_v[pl.ds(i * 16, 16)]
        plsc.addupdate_scatter(deg_v, [idx], ones16)
        return carry

    lax.fori_loop(0, EW // 16, body, 0)
    oof = pl.multiple_of(w * N, 8)
    pltpu.sync_copy(deg_v, out_hbm.at[pl.ds(oof, N)])


@functools.partial(
    pl.kernel,
    out_type=jax.ShapeDtypeStruct((NC, N, D), jnp.float32),
    mesh=_mesh,
    compiler_params=pltpu.CompilerParams(needs_layout_passes=False),
    scratch_types=[
        [pltpu.VMEM((2 * NBUF, CH), jnp.int32)] * 2,
        pltpu.VMEM((TAILE,), jnp.int32),
        pltpu.VMEM((TAILE,), jnp.int32),
        [pltpu.VMEM((CH, D), jnp.float32)] * NBUF,
        pltpu.VMEM_SHARED((N, D), jnp.float32),
        [pltpu.SemaphoreType.DMA] * NBUF,
        [pltpu.SemaphoreType.DMA] * NBUF,
        [pltpu.SemaphoreType.DMA] * 2,
    ],
)
def _edge_kernel(hp_hbm, il_hbm, src_hbm, dst_hbm, out_hbm,
                 ibuf, sidx_t, didx_t, rows, acc, gsem, ssem, isem):
    c = lax.axis_index("c")
    s = lax.axis_index("s")
    w = s * NC + c
    ebase = pl.multiple_of(w * EW, 8)
    rbase = s * RPT
    ilbase = w * NITER

    zeros16 = jnp.zeros((16,), jnp.float32)

    def zr(i, carry):
        rows[0][i // 8, pl.ds((i % 8) * 16, 16)] = zeros16
        return carry

    lax.fori_loop(0, CH * 8, zr, 0)

    def zacc(k, carry):
        pltpu.sync_copy(rows[0], acc.at[pl.ds(rbase + k * CH, CH)])
        return carry

    lax.fori_loop(0, RPT // CH, zacc, 0)
    rem = RPT - (RPT // CH) * CH
    if rem:
        pltpu.sync_copy(rows[0].at[pl.ds(0, rem)],
                        acc.at[pl.ds(rbase + RPT - rem, rem)])

    @pl.when(s == NS - 1)
    def _zero_tail():
        pltpu.sync_copy(rows[0].at[pl.ds(0, TAIL)],
                        acc.at[pl.ds(NS * RPT, TAIL)])

    plsc.subcore_barrier()

    def start_gather(b, buf):
        pltpu.async_copy(hp_hbm.at[buf.at[2 * b]], rows[b], gsem[b])

    def wait_gather(b):
        pltpu.make_async_copy(hp_hbm.at[ibuf[0].at[2 * b]], rows[b],
                              gsem[b]).wait()

    def start_scatter(b, buf):
        pltpu.async_copy(rows[b], acc.at[buf.at[2 * b + 1]], ssem[b],
                         add=True)

    def wait_scatter(b):
        pltpu.make_async_copy(rows[b], acc.at[ibuf[0].at[1]], ssem[b]).wait()

    def wait_iblock(p):
        pltpu.make_async_copy(il_hbm.at[ilbase], ibuf[p], isem[p]).wait()

    def one_iter(t, p, last):
        for b in range(NBUF):
            wait_gather(b)
            start_scatter(b, ibuf[p])
        if not last:
            wait_iblock(1 - p)
        for b in range(NBUF):
            wait_scatter(b)
            if not last:
                start_gather(b, ibuf[1 - p])
        if not last:
            blk = jnp.minimum(t + 2, NITER - 1)
            pltpu.async_copy(il_hbm.at[ilbase + blk], ibuf[p], isem[p])

    pltpu.sync_copy(il_hbm.at[ilbase], ibuf[0])
    for b in range(NBUF):
        start_gather(b, ibuf[0])
    pltpu.async_copy(il_hbm.at[ilbase + 1], ibuf[1], isem[1])

    def body(k, carry):
        one_iter(2 * k, 0, False)
        one_iter(2 * k + 1, 1, False)
        return carry

    lax.fori_loop(0, NITER // 2 - 1, body, 0)
    one_iter(NITER - 2, 0, False)
    one_iter(NITER - 1, 1, True)
    wait_iblock(0)

    toff = pl.multiple_of(ebase + NCHUNK * CH, 8)
    pltpu.sync_copy(src_hbm.at[pl.ds(toff, TAILE)], sidx_t)
    pltpu.async_copy(hp_hbm.at[sidx_t], rows[0].at[pl.ds(0, TAILE)],
                     gsem[0]).wait()
    pltpu.sync_copy(dst_hbm.at[pl.ds(toff, TAILE)], didx_t)
    pltpu.sync_copy(rows[0].at[pl.ds(0, TAILE)], acc.at[didx_t], add=True)

    plsc.subcore_barrier()
    pltpu.sync_copy(acc.at[pl.ds(rbase, RPT)],
                    out_hbm.at[c, pl.ds(rbase, RPT)])

    @pl.when(s == NS - 1)
    def _copy_tail():
        pltpu.sync_copy(acc.at[pl.ds(NS * RPT, TAIL)],
                        out_hbm.at[c, pl.ds(NS * RPT, TAIL)])


def _prep_body(x_ref, w_ref, degp_ref, hp_ref, dis_ref):
    deg = jnp.sum(degp_ref[...], axis=0)
    dis = jnp.where(deg > 0, lax.rsqrt(jnp.maximum(deg, 1e-12)), 0.0)
    h = jnp.dot(x_ref[...], w_ref[...], preferred_element_type=jnp.float32)
    hp_ref[...] = h * dis[:, None]
    dis_ref[...] = dis[:, None]


def _tc_prep(x, W, degp):
    return pl.pallas_call(
        _prep_body,
        out_shape=[
            jax.ShapeDtypeStruct((N, D), jnp.float32),
            jax.ShapeDtypeStruct((N, 1), jnp.float32),
        ],
    )(x, W, degp)


def _final_body(p0_ref, p1_ref, dis_ref, b_ref, w2_ref, b2_ref, out_ref):
    agg = (p0_ref[...] + p1_ref[...]) * dis_ref[...]
    hidden = jnp.maximum(agg + b_ref[...], 0.0)
    out_ref[...] = (jnp.sum(hidden * w2_ref[...], axis=1, keepdims=True)
                    + b2_ref[0, 0])


def _tc_final(p0, p1, dis, b, w2, b2):
    return pl.pallas_call(
        _final_body,
        grid=(N // RB,),
        in_specs=[
            pl.BlockSpec((RB, D), lambda i: (i, 0)),
            pl.BlockSpec((RB, D), lambda i: (i, 0)),
            pl.BlockSpec((RB, 1), lambda i: (i, 0)),
            pl.BlockSpec((1, D), lambda i: (0, 0)),
            pl.BlockSpec((1, D), lambda i: (0, 0)),
            pl.BlockSpec((1, 1), lambda i: (0, 0)),
        ],
        out_specs=pl.BlockSpec((RB, 1), lambda i: (i, 0)),
        out_shape=jax.ShapeDtypeStruct((N, 1), jnp.float32),
    )(p0, p1, dis, b, w2, b2)


def kernel(x, edge_index, W, b, W2, b2):
    ei = edge_index.astype(jnp.int32)
    src = ei[0]
    dst = ei[1]
    main = ei.reshape(2, NW, EW)[:, :, :NCHUNK * CH]
    main = main.reshape(2, NW, NITER, NBUF, CH)
    il = jnp.transpose(main, (1, 2, 3, 0, 4)).reshape(NW * NITER,
                                                      2 * NBUF, CH)
    degp = _deg_kernel(dst).reshape(NW, N)
    hp, dis = _tc_prep(x, W, degp)
    paggs = _edge_kernel(hp, il, src, dst)
    return _tc_final(paggs[0], paggs[1], dis,
                     b.reshape(1, D), W2.reshape(1, D), b2.reshape(1, 1))
